# baseline (speedup 1.0000x reference)
import functools

import jax
import jax.numpy as jnp
from jax import lax
from jax.experimental import pallas as pl
from jax.experimental.pallas import tpu as pltpu
from jax.experimental.pallas import tpu_sc as plsc

N = 10000
E = 320000
D = 128
NP = 10240
NC, NS = 2, 16
NW = NC * NS
CHUNK = 128
K = 80
EP = NW * K * CHUNK
ROWS_PER_TILE = NP // NS


def _sc_mesh():
    return plsc.VectorSubcoreMesh(core_axis_name="c", subcore_axis_name="s")


def _deg_body(didx_hbm, zeros1_hbm, out_hbm, idx_v, ones_v, sem0, sem1, dacc):
    c = lax.axis_index("c")
    s = lax.axis_index("s")
    wid = s * NC + c

    pltpu.sync_copy(didx_hbm.at[wid], idx_v)
    for i in range(8):
        ones_v[pl.ds(16 * i, 16)] = jnp.ones((16,), jnp.float32)
    pltpu.sync_copy(zeros1_hbm.at[pl.ds(s * ROWS_PER_TILE, ROWS_PER_TILE)],
                    dacc.at[pl.ds(s * ROWS_PER_TILE, ROWS_PER_TILE)])
    plsc.subcore_barrier()

    def body(j, _):
        a = 2 * j
        pltpu.async_copy(ones_v, dacc.at[idx_v.at[a]], sem0, add=True)
        pltpu.async_copy(ones_v, dacc.at[idx_v.at[a + 1]], sem1, add=True)
        pltpu.make_async_copy(ones_v, dacc.at[idx_v.at[a]], sem0).wait()
        pltpu.make_async_copy(ones_v, dacc.at[idx_v.at[a + 1]], sem1).wait()
        return 0

    lax.fori_loop(0, K // 2, body, 0)
    plsc.subcore_barrier()
    pltpu.sync_copy(dacc.at[pl.ds(s * ROWS_PER_TILE, ROWS_PER_TILE)],
                    out_hbm.at[c, pl.ds(s * ROWS_PER_TILE, ROWS_PER_TILE)])


def _make_deg_kernel():
    return pl.kernel(
        _deg_body,
        out_type=jax.ShapeDtypeStruct((NC, NP), jnp.float32),
        mesh=_sc_mesh(),
        scratch_types=[
            pltpu.VMEM((K, CHUNK), jnp.int32),
            pltpu.VMEM((CHUNK,), jnp.float32),
            pltpu.SemaphoreType.DMA,
            pltpu.SemaphoreType.DMA,
            pltpu.VMEM_SHARED((NP,), jnp.float32),
        ],
    )


G = 8
NG = K // G

DH = D // NC
K2 = EP // (NS * CHUNK)
NG2 = K2 // G
NB = 8
AHEAD = 5


def _agg_body(g_hbm, idx_hbm, zeros_hbm, out_hbm, *scr):
    ig = scr[0]
    bufs = scr[1:1 + NB]
    gsems = scr[1 + NB:1 + 2 * NB]
    ssems = scr[1 + 2 * NB:1 + 3 * NB]
    isem = scr[1 + 3 * NB]
    acc = scr[2 + 3 * NB]
    c = lax.axis_index("c")
    s = lax.axis_index("s")

    pltpu.sync_copy(zeros_hbm.at[pl.ds(s * ROWS_PER_TILE, ROWS_PER_TILE)],
                    acc.at[pl.ds(s * ROWS_PER_TILE, ROWS_PER_TILE)])
    pltpu.sync_copy(idx_hbm.at[c, s, 0], ig.at[0])
    plsc.subcore_barrier()
    for t in range(AHEAD):
        pltpu.async_copy(g_hbm.at[ig.at[0, t, 0]], bufs[t], gsems[t])
    pltpu.async_copy(idx_hbm.at[c, s, 1], ig.at[1], isem)

    def emit_iter(r, rnext, t, first_group, last_group):
        if t == 3 and not last_group:
            pltpu.make_async_copy(idx_hbm.at[c, s, 0], ig.at[0], isem).wait()
        pltpu.make_async_copy(g_hbm.at[ig.at[r, t, 0]], bufs[t],
                              gsems[t]).wait()
        pltpu.async_copy(bufs[t], acc.at[ig.at[r, t, 1]], ssems[t], add=True)
        if not (last_group and t >= G - AHEAD):
            sub = (t + AHEAD) % NB
            slot = r if t < G - AHEAD else rnext
            if not (first_group and t < G - AHEAD):
                pltpu.make_async_copy(bufs[sub], acc.at[ig.at[r, t, 1]],
                                      ssems[sub]).wait()
            pltpu.async_copy(g_hbm.at[ig.at[slot, sub, 0]], bufs[sub],
                             gsems[sub])

    for t in range(G):
        emit_iter(0, 1, t, True, False)
    pltpu.async_copy(idx_hbm.at[c, s, 2], ig.at[2], isem)

    def body(gi, _):
        r = gi % 3
        rnext = (gi + 1) % 3
        for t in range(G):
            emit_iter(r, rnext, t, False, False)

        @pl.when(gi < NG2 - 2)
        def _():
            pltpu.async_copy(idx_hbm.at[c, s, gi + 2], ig.at[(gi + 2) % 3],
                             isem)
        return 0

    lax.fori_loop(1, NG2 - 1, body, 0)

    rl = (NG2 - 1) % 3
    for t in range(G):
        emit_iter(rl, 0, t, False, True)
    for t in range(NB):
        pltpu.make_async_copy(bufs[t], acc.at[ig.at[rl, t % G, 1]],
                              ssems[t]).wait()

    plsc.subcore_barrier()
    pltpu.sync_copy(acc.at[pl.ds(s * ROWS_PER_TILE, ROWS_PER_TILE)],
                    out_hbm.at[pl.ds(s * ROWS_PER_TILE, ROWS_PER_TILE), c])


def _make_agg_kernel():
    scratch = [pltpu.VMEM((3, G, 2, CHUNK), jnp.int32)]
    scratch += [pltpu.VMEM((CHUNK, DH), jnp.float32) for _ in range(NB)]
    scratch += [pltpu.SemaphoreType.DMA for _ in range(2 * NB)]
    scratch += [pltpu.SemaphoreType.DMA]
    scratch += [pltpu.VMEM_SHARED((NP, DH), jnp.float32)]
    return pl.kernel(
        _agg_body,
        out_type=jax.ShapeDtypeStruct((NP, NC, DH), jnp.float32),
        mesh=_sc_mesh(),
        scratch_types=scratch,
        compiler_params=pltpu.CompilerParams(use_tc_tiling_on_sc=False),
    )


_BLK = 1024
_GRID = NP // _BLK


def _b1_body(x_ref, w_ref, degp_ref, g_ref):
    dp = degp_ref[...]
    dinv = lax.rsqrt(dp[0] + dp[1] + 1.0)
    h = jnp.dot(x_ref[...], w_ref[...], preferred_element_type=jnp.float32)
    g_ref[...] = h * dinv


def _b1(x_pad, W1, degp):
    return pl.pallas_call(
        _b1_body,
        grid=(_GRID,),
        in_specs=[
            pl.BlockSpec((_BLK, D), lambda i: (i, 0)),
            pl.BlockSpec((D, D), lambda i: (0, 0)),
            pl.BlockSpec((NC, _BLK, 1), lambda i: (0, i, 0)),
        ],
        out_specs=pl.BlockSpec((_BLK, D), lambda i: (i, 0)),
        out_shape=jax.ShapeDtypeStruct((NP, D), jnp.float32),
    )(x_pad, W1, degp)


def _b2_body(s_ref, g1_ref, degp_ref, b_ref, w_ref, g2_ref):
    dp = degp_ref[...]
    dinv = lax.rsqrt(dp[0] + dp[1] + 1.0)
    z = (s_ref[...] + g1_ref[...]) * dinv + b_ref[...]
    h = jnp.maximum(z, 0.0)
    g2_ref[...] = jnp.dot(h, w_ref[...], preferred_element_type=jnp.float32) * dinv


def _b2(S, g1, degp, b1r, W2):
    return pl.pallas_call(
        _b2_body,
        grid=(_GRID,),
        in_specs=[
            pl.BlockSpec((_BLK, D), lambda i: (i, 0)),
            pl.BlockSpec((_BLK, D), lambda i: (i, 0)),
            pl.BlockSpec((NC, _BLK, 1), lambda i: (0, i, 0)),
            pl.BlockSpec((1, D), lambda i: (0, 0)),
            pl.BlockSpec((D, D), lambda i: (0, 0)),
        ],
        out_specs=pl.BlockSpec((_BLK, D), lambda i: (i, 0)),
        out_shape=jax.ShapeDtypeStruct((NP, D), jnp.float32),
    )(S, g1, degp, b1r, W2)


def _b3_body(s_ref, g2_ref, degp_ref, b_ref, wo_ref, bo_ref, o_ref):
    dp = degp_ref[...]
    dinv = lax.rsqrt(dp[0] + dp[1] + 1.0)
    z = (s_ref[...] + g2_ref[...]) * dinv + b_ref[...]
    h = jnp.maximum(z, 0.0)
    o_ref[...] = jnp.dot(h, wo_ref[...], preferred_element_type=jnp.float32) + bo_ref[...]


def _b3(S, g2, degp, b2r, Wo_p, bo_p):
    return pl.pallas_call(
        _b3_body,
        grid=(_GRID,),
        in_specs=[
            pl.BlockSpec((_BLK, D), lambda i: (i, 0)),
            pl.BlockSpec((_BLK, D), lambda i: (i, 0)),
            pl.BlockSpec((NC, _BLK, 1), lambda i: (0, i, 0)),
            pl.BlockSpec((1, D), lambda i: (0, 0)),
            pl.BlockSpec((D, 8), lambda i: (0, 0)),
            pl.BlockSpec((1, 8), lambda i: (0, 0)),
        ],
        out_specs=pl.BlockSpec((_BLK, 8), lambda i: (i, 0)),
        out_shape=jax.ShapeDtypeStruct((NP, 8), jnp.float32),
    )(S, g2, degp, b2r, Wo_p, bo_p)


def kernel(x, edge_index, W1, b1, W2, b2, Wo, bo):
    ei = edge_index.astype(jnp.int32)
    pad = jnp.full((2, EP - E), N, jnp.int32)
    eip = jnp.concatenate([ei, pad], axis=1)
    src_r = eip[0].reshape(NS, NG2, G, CHUNK)
    dst_r = eip[1].reshape(NS, NG2, G, CHUNK)
    idx = jnp.stack(
        [jnp.stack([2 * src_r + c, dst_r], axis=3) for c in range(NC)],
        axis=0)
    didx_deg = eip[1].reshape(NW, K, CHUNK)

    x_pad = jnp.zeros((NP, D), jnp.float32).at[:N].set(x)
    zeros1 = jnp.zeros((NP,), jnp.float32)
    zerosh = jnp.zeros((NP, DH), jnp.float32)

    degp = _make_deg_kernel()(didx_deg, zeros1)
    degp = degp.reshape(NC, NP, 1)

    g1 = _b1(x_pad, W1, degp)

    agg = _make_agg_kernel()
    s1 = agg(g1.reshape(2 * NP, DH), idx, zerosh)
    g2 = _b2(s1.reshape(NP, D), g1, degp, b1.reshape(1, D), W2)

    s2 = agg(g2.reshape(2 * NP, DH), idx, zerosh)
    Wo_p = jnp.zeros((D, 8), jnp.float32).at[:, :1].set(Wo)
    bo_p = jnp.zeros((1, 8), jnp.float32).at[0, 0].set(bo[0])
    out = _b3(s2.reshape(NP, D), g2, degp, b2.reshape(1, D), Wo_p, bo_p)
    return out[:N, :1]


# speedup vs baseline: 1.0946x; 1.0946x over previous
import functools

import jax
import jax.numpy as jnp
from jax import lax
from jax.experimental import pallas as pl
from jax.experimental.pallas import tpu as pltpu
from jax.experimental.pallas import tpu_sc as plsc

N = 10000
E = 320000
D = 128
NP = 10240
NC, NS = 2, 16
NW = NC * NS
CHUNK = 128
K = 80
EP = NW * K * CHUNK
ROWS_PER_TILE = NP // NS


def _sc_mesh():
    return plsc.VectorSubcoreMesh(core_axis_name="c", subcore_axis_name="s")


def _deg_body(didx_hbm, zeros1_hbm, out_hbm, idx_v, ones_v, sem0, sem1, dacc):
    c = lax.axis_index("c")
    s = lax.axis_index("s")
    wid = s * NC + c

    pltpu.sync_copy(didx_hbm.at[wid], idx_v)
    for i in range(8):
        ones_v[pl.ds(16 * i, 16)] = jnp.ones((16,), jnp.float32)
    pltpu.sync_copy(zeros1_hbm.at[pl.ds(s * ROWS_PER_TILE, ROWS_PER_TILE)],
                    dacc.at[pl.ds(s * ROWS_PER_TILE, ROWS_PER_TILE)])
    plsc.subcore_barrier()

    def body(j, _):
        a = 2 * j
        pltpu.async_copy(ones_v, dacc.at[idx_v.at[a]], sem0, add=True)
        pltpu.async_copy(ones_v, dacc.at[idx_v.at[a + 1]], sem1, add=True)
        pltpu.make_async_copy(ones_v, dacc.at[idx_v.at[a]], sem0).wait()
        pltpu.make_async_copy(ones_v, dacc.at[idx_v.at[a + 1]], sem1).wait()
        return 0

    lax.fori_loop(0, K // 2, body, 0)
    plsc.subcore_barrier()
    pltpu.sync_copy(dacc.at[pl.ds(s * ROWS_PER_TILE, ROWS_PER_TILE)],
                    out_hbm.at[c, pl.ds(s * ROWS_PER_TILE, ROWS_PER_TILE)])


def _make_deg_kernel():
    return pl.kernel(
        _deg_body,
        out_type=jax.ShapeDtypeStruct((NC, NP), jnp.float32),
        mesh=_sc_mesh(),
        scratch_types=[
            pltpu.VMEM((K, CHUNK), jnp.int32),
            pltpu.VMEM((CHUNK,), jnp.float32),
            pltpu.SemaphoreType.DMA,
            pltpu.SemaphoreType.DMA,
            pltpu.VMEM_SHARED((NP,), jnp.float32),
        ],
    )


G = 8
NG = K // G

HEAVY = 0
KH = 128
KL = 32
NGH = KH // G
NGL = KL // G
EH = NS * KH * CHUNK
EL = NS * KL * CHUNK


def _agg_loop(g_hbm, idx_hbm, s, ng, ig, buf0, buf1, isem, gsem0, gsem1, acc):
    bufs = (buf0, buf1)
    gsems = (gsem0, gsem1)
    pltpu.sync_copy(idx_hbm.at[s, 0], ig.at[0])
    plsc.subcore_barrier()
    pltpu.async_copy(g_hbm.at[ig.at[0, 0, 0]], buf0, gsem0)
    pltpu.async_copy(idx_hbm.at[s, 1], ig.at[1], isem)

    def body(gi, _):
        r = gi % 2
        for t in range(G):
            p = t % 2
            if t < G - 1:
                pltpu.async_copy(g_hbm.at[ig.at[r, t + 1, 0]],
                                 bufs[1 - p], gsems[1 - p])
            else:
                @pl.when(gi < ng - 1)
                def _():
                    pltpu.make_async_copy(idx_hbm.at[s, 0], ig.at[0],
                                          isem).wait()
                    pltpu.async_copy(g_hbm.at[ig.at[1 - r, 0, 0]],
                                     bufs[1 - p], gsems[1 - p])
            pltpu.make_async_copy(g_hbm.at[ig.at[r, t, 0]],
                                  bufs[p], gsems[p]).wait()
            pltpu.sync_copy(bufs[p], acc.at[ig.at[r, t, 1]], add=True)

        @pl.when(gi < ng - 2)
        def _():
            pltpu.async_copy(idx_hbm.at[s, gi + 2], ig.at[r], isem)
        return 0

    lax.fori_loop(0, ng, body, 0)


def _agg_body(g_hbm, idxh_hbm, idxl_hbm, zeros_hbm, out_hbm,
              ig, buf0, buf1, isem, gsem0, gsem1, acc):
    c = lax.axis_index("c")
    s = lax.axis_index("s")

    pltpu.sync_copy(zeros_hbm.at[pl.ds(s * ROWS_PER_TILE, ROWS_PER_TILE)],
                    acc.at[pl.ds(s * ROWS_PER_TILE, ROWS_PER_TILE)])

    @pl.when(c == HEAVY)
    def _():
        _agg_loop(g_hbm, idxh_hbm, s, NGH, ig, buf0, buf1, isem,
                  gsem0, gsem1, acc)

    @pl.when(c != HEAVY)
    def _():
        _agg_loop(g_hbm, idxl_hbm, s, NGL, ig, buf0, buf1, isem,
                  gsem0, gsem1, acc)

    plsc.subcore_barrier()
    pltpu.sync_copy(acc.at[pl.ds(s * ROWS_PER_TILE, ROWS_PER_TILE)],
                    out_hbm.at[c, pl.ds(s * ROWS_PER_TILE, ROWS_PER_TILE)])


def _make_agg_kernel():
    return pl.kernel(
        _agg_body,
        out_type=jax.ShapeDtypeStruct((NC, NP, D), jnp.float32),
        mesh=_sc_mesh(),
        scratch_types=[
            pltpu.VMEM((2, G, 2, CHUNK), jnp.int32),
            pltpu.VMEM((CHUNK, D), jnp.float32),
            pltpu.VMEM((CHUNK, D), jnp.float32),
            pltpu.SemaphoreType.DMA,
            pltpu.SemaphoreType.DMA,
            pltpu.SemaphoreType.DMA,
            pltpu.VMEM_SHARED((NP, D), jnp.float32),
        ],
    )


_BLK = 1024
_GRID = NP // _BLK


def _b1_body(x_ref, w_ref, degp_ref, g_ref):
    dp = degp_ref[...]
    dinv = lax.rsqrt(dp[0] + dp[1] + 1.0)
    h = jnp.dot(x_ref[...], w_ref[...], preferred_element_type=jnp.float32)
    g_ref[...] = h * dinv


def _b1(x_pad, W1, degp):
    return pl.pallas_call(
        _b1_body,
        grid=(_GRID,),
        in_specs=[
            pl.BlockSpec((_BLK, D), lambda i: (i, 0)),
            pl.BlockSpec((D, D), lambda i: (0, 0)),
            pl.BlockSpec((NC, _BLK, 1), lambda i: (0, i, 0)),
        ],
        out_specs=pl.BlockSpec((_BLK, D), lambda i: (i, 0)),
        out_shape=jax.ShapeDtypeStruct((NP, D), jnp.float32),
    )(x_pad, W1, degp)


def _b2_body(p_ref, g1_ref, degp_ref, b_ref, w_ref, g2_ref):
    dp = degp_ref[...]
    dinv = lax.rsqrt(dp[0] + dp[1] + 1.0)
    p = p_ref[...]
    z = (p[0] + p[1] + g1_ref[...]) * dinv + b_ref[...]
    h = jnp.maximum(z, 0.0)
    g2_ref[...] = jnp.dot(h, w_ref[...], preferred_element_type=jnp.float32) * dinv


def _b2(partials, g1, degp, b1r, W2):
    return pl.pallas_call(
        _b2_body,
        grid=(_GRID,),
        in_specs=[
            pl.BlockSpec((NC, _BLK, D), lambda i: (0, i, 0)),
            pl.BlockSpec((_BLK, D), lambda i: (i, 0)),
            pl.BlockSpec((NC, _BLK, 1), lambda i: (0, i, 0)),
            pl.BlockSpec((1, D), lambda i: (0, 0)),
            pl.BlockSpec((D, D), lambda i: (0, 0)),
        ],
        out_specs=pl.BlockSpec((_BLK, D), lambda i: (i, 0)),
        out_shape=jax.ShapeDtypeStruct((NP, D), jnp.float32),
    )(partials, g1, degp, b1r, W2)


def _b3_body(p_ref, g2_ref, degp_ref, b_ref, wo_ref, bo_ref, o_ref):
    dp = degp_ref[...]
    dinv = lax.rsqrt(dp[0] + dp[1] + 1.0)
    p = p_ref[...]
    z = (p[0] + p[1] + g2_ref[...]) * dinv + b_ref[...]
    h = jnp.maximum(z, 0.0)
    o_ref[...] = jnp.dot(h, wo_ref[...], preferred_element_type=jnp.float32) + bo_ref[...]


def _b3(partials, g2, degp, b2r, Wo_p, bo_p):
    return pl.pallas_call(
        _b3_body,
        grid=(_GRID,),
        in_specs=[
            pl.BlockSpec((NC, _BLK, D), lambda i: (0, i, 0)),
            pl.BlockSpec((_BLK, D), lambda i: (i, 0)),
            pl.BlockSpec((NC, _BLK, 1), lambda i: (0, i, 0)),
            pl.BlockSpec((1, D), lambda i: (0, 0)),
            pl.BlockSpec((D, 8), lambda i: (0, 0)),
            pl.BlockSpec((1, 8), lambda i: (0, 0)),
        ],
        out_specs=pl.BlockSpec((_BLK, 8), lambda i: (i, 0)),
        out_shape=jax.ShapeDtypeStruct((NP, 8), jnp.float32),
    )(partials, g2, degp, b2r, Wo_p, bo_p)


def kernel(x, edge_index, W1, b1, W2, b2, Wo, bo):
    ei = edge_index.astype(jnp.int32)
    pad = jnp.full((2, EP - E), N, jnp.int32)
    eip = jnp.concatenate([ei, pad], axis=1)
    idxh = jnp.stack([eip[0, :EH].reshape(NS, NGH, G, CHUNK),
                      eip[1, :EH].reshape(NS, NGH, G, CHUNK)], axis=3)
    idxl = jnp.stack([eip[0, EH:].reshape(NS, NGL, G, CHUNK),
                      eip[1, EH:].reshape(NS, NGL, G, CHUNK)], axis=3)
    didx_deg = eip[1].reshape(NW, K, CHUNK)

    x_pad = jnp.zeros((NP, D), jnp.float32).at[:N].set(x)
    zeros1 = jnp.zeros((NP,), jnp.float32)
    zeros2 = jnp.zeros((NP, D), jnp.float32)

    degp = _make_deg_kernel()(didx_deg, zeros1)
    degp = degp.reshape(NC, NP, 1)

    g1 = _b1(x_pad, W1, degp)

    agg = _make_agg_kernel()
    p1 = agg(g1, idxh, idxl, zeros2)
    g2 = _b2(p1, g1, degp, b1.reshape(1, D), W2)

    p2 = agg(g2, idxh, idxl, zeros2)
    Wo_p = jnp.zeros((D, 8), jnp.float32).at[:, :1].set(Wo)
    bo_p = jnp.zeros((1, 8), jnp.float32).at[0, 0].set(bo[0])
    out = _b3(p2, g2, degp, b2.reshape(1, D), Wo_p, bo_p)
    return out[:N, :1]
